# baseline (speedup 1.0000x reference)
import functools
import math

import jax
import jax.numpy as jnp
from jax import lax
from jax.experimental import pallas as pl
from jax.experimental.pallas import tpu as pltpu
from jax.experimental.pallas import tpu_sc as plsc

K = 64
MV_BLK = 16384


def _matvec_body(x_ref, w_ref, m_ref, n_ref, o_ref):
    s = lax.dot_general(
        x_ref[...], w_ref[...],
        (((1,), (0,)), ((), ())),
        preferred_element_type=jnp.float32,
    )
    o_ref[...] = s / n_ref[0, 0] + m_ref[...]


NLEV = 8
GBIG = 1 << 24


def _topk_body(s_ref, gi_ref, tv_ref, work_ref):
    b, c, l = work_ref.shape
    g = c * l
    work_ref[...] = s_ref[...]
    siota = lax.broadcasted_iota(jnp.int32, (b, c, l), 1)
    liota2 = lax.broadcasted_iota(jnp.int32, (b, l), 1)
    base = lax.broadcasted_iota(jnp.int32, (b, K), 0) * g
    kiota = lax.broadcasted_iota(jnp.int32, (b, K), 1)
    neg = -jnp.inf

    cv, cg = [], []
    for _ in range(NLEV):
        w3 = work_ref[...]
        mx = jnp.max(w3, axis=1, keepdims=True)
        sidx = jnp.min(jnp.where(w3 == mx, siota, c), axis=1, keepdims=True)
        work_ref[...] = jnp.where(siota == sidx, neg, w3)
        cv.append(mx[:, 0, :])
        cg.append(sidx[:, 0, :] * l + liota2)

    def body(i, carry):
        cvs = list(carry[:NLEV])
        cgs = list(carry[NLEV:2 * NLEV])
        colcnt, vacc, iacc = carry[2 * NLEV:]
        cmb = cvs[0]
        for r in range(1, NLEV):
            cmb = jnp.maximum(cmb, cvs[r])
        mx = jnp.max(cmb, axis=1, keepdims=True)
        m8 = jnp.full((b, l), GBIG, jnp.int32)
        for r in range(NLEV):
            m8 = jnp.minimum(m8, jnp.where(cvs[r] == mx, cgs[r], GBIG))
        gmin = jnp.min(m8, axis=1, keepdims=True)
        for r in range(NLEV):
            cvs[r] = jnp.where(cgs[r] == gmin, neg, cvs[r])
        lanecol = gmin - (gmin // l) * l
        colcnt = colcnt + jnp.where(liota2 == lanecol, 1, 0)
        sel = kiota == i
        vacc = jnp.where(sel, mx, vacc)
        iacc = jnp.where(sel, gmin, iacc)
        return tuple(cvs) + tuple(cgs) + (colcnt, vacc, iacc)

    init = tuple(cv) + tuple(cg) + (
        jnp.zeros((b, l), jnp.int32),
        jnp.zeros((b, K), jnp.float32),
        jnp.zeros((b, K), jnp.int32),
    )
    res = lax.fori_loop(0, K, body, init, unroll=8)
    colcnt, vacc, iacc = res[2 * NLEV:]
    gi_ref[...] = iacc + base
    tv_ref[...] = jnp.tanh(vacc)

    overflow = jnp.max(colcnt) >= NLEV

    @pl.when(overflow)
    def _():
        work_ref[...] = s_ref[...]
        giota = siota * l + lax.broadcasted_iota(jnp.int32, (b, c, l), 2)

        def sbody(i, carry):
            vacc2, iacc2 = carry
            x = work_ref[...]
            mx = jnp.max(jnp.max(x, axis=1, keepdims=True), axis=2,
                         keepdims=True)
            gidx = jnp.where(x == mx, giota, GBIG)
            idxm = jnp.min(jnp.min(gidx, axis=1, keepdims=True), axis=2,
                           keepdims=True)
            work_ref[...] = jnp.where(giota == idxm, neg, x)
            sel = kiota == i
            vacc2 = jnp.where(sel, mx[:, :, 0], vacc2)
            iacc2 = jnp.where(sel, idxm[:, :, 0], iacc2)
            return vacc2, iacc2

        vacc2, iacc2 = lax.fori_loop(
            0, K, sbody,
            (jnp.zeros((b, K), jnp.float32), jnp.zeros((b, K), jnp.int32)),
        )
        gi_ref[...] = iacc2 + base
        tv_ref[...] = jnp.tanh(vacc2)


def _sc_gather(emb_hbm, gi_hbm, out_hbm, idx_v, rows_v, sem):
    num_b = out_hbm.shape[0]
    nc = 2
    wid = lax.axis_index("s") * nc + lax.axis_index("c")

    @pl.when(wid < num_b)
    def _():
        pltpu.sync_copy(gi_hbm.at[wid], idx_v)
        pltpu.async_copy(emb_hbm.at[idx_v], rows_v, sem).wait()
        pltpu.sync_copy(rows_v, out_hbm.at[wid])


def _scale_t_body(x_ref, tv_ref, o_ref):
    xt = jnp.transpose(x_ref[0], (1, 0))
    o_ref[0] = xt * tv_ref[0]


def kernel(node_embs, mask, scorer):
    b, g, f = node_embs.shape
    rows = b * g
    nrm = jnp.linalg.norm(scorer).reshape(1, 1)

    emb_flat = node_embs.reshape(rows, f)
    scores = pl.pallas_call(
        _matvec_body,
        grid=(rows // MV_BLK,),
        in_specs=[
            pl.BlockSpec((MV_BLK, f), lambda i: (i, 0)),
            pl.BlockSpec((f, 1), lambda i: (0, 0)),
            pl.BlockSpec((MV_BLK, 1), lambda i: (i, 0)),
            pl.BlockSpec((1, 1), lambda i: (0, 0)),
        ],
        out_specs=pl.BlockSpec((MV_BLK, 1), lambda i: (i, 0)),
        out_shape=jax.ShapeDtypeStruct((rows, 1), jnp.float32),
    )(emb_flat, scorer, mask.reshape(rows, 1), nrm)

    nc = g // 128
    gidx, tvals = pl.pallas_call(
        _topk_body,
        in_specs=[
            pl.BlockSpec((b, nc, 128), lambda: (0, 0, 0)),
        ],
        out_specs=[
            pl.BlockSpec((b, K), lambda: (0, 0)),
            pl.BlockSpec((b, K), lambda: (0, 0)),
        ],
        out_shape=[
            jax.ShapeDtypeStruct((b, K), jnp.int32),
            jax.ShapeDtypeStruct((b, K), jnp.float32),
        ],
        scratch_shapes=[pltpu.VMEM((b, nc, 128), jnp.float32)],
    )(scores.reshape(b, nc, 128))

    mesh = plsc.VectorSubcoreMesh(core_axis_name="c", subcore_axis_name="s")
    gathered = pl.kernel(
        _sc_gather,
        out_type=jax.ShapeDtypeStruct((b, K, f), jnp.float32),
        mesh=mesh,
        scratch_types=[
            pltpu.VMEM((K,), jnp.int32),
            pltpu.VMEM((K, f), jnp.float32),
            pltpu.SemaphoreType.DMA,
        ],
    )(emb_flat, gidx)

    out = pl.pallas_call(
        _scale_t_body,
        grid=(b,),
        in_specs=[
            pl.BlockSpec((1, K, f), lambda i: (i, 0, 0)),
            pl.BlockSpec((1, 1, K), lambda i: (i, 0, 0)),
        ],
        out_specs=pl.BlockSpec((1, f, K), lambda i: (i, 0, 0)),
        out_shape=jax.ShapeDtypeStruct((b, f, K), jnp.float32),
    )(gathered, tvals.reshape(b, 1, K))
    return out


# speedup vs baseline: 1.6041x; 1.6041x over previous
import functools
import math

import jax
import jax.numpy as jnp
from jax import lax
from jax.experimental import pallas as pl
from jax.experimental.pallas import tpu as pltpu
from jax.experimental.pallas import tpu_sc as plsc

K = 64
MV_BLK = 16384


def _matvec_body(x_ref, w_ref, o_ref):
    o_ref[...] = lax.dot_general(
        x_ref[...], w_ref[...],
        (((1,), (0,)), ((), ())),
        preferred_element_type=jnp.float32,
    )


NLEV = 8
GBIG = 1 << 24


def _topk_body(s_ref, m_ref, n_ref, gi_ref, tv_ref, work_ref):
    b, c, l = work_ref.shape
    g = c * l
    work_ref[...] = s_ref[...] / n_ref[0, 0] + m_ref[...]
    siota = lax.broadcasted_iota(jnp.int32, (b, c, l), 1)
    liota2 = lax.broadcasted_iota(jnp.int32, (b, l), 1)
    base = lax.broadcasted_iota(jnp.int32, (b, K), 0) * g
    kiota = lax.broadcasted_iota(jnp.int32, (b, K), 1)
    neg = -jnp.inf

    cv, cg = [], []
    for _ in range(NLEV):
        w3 = work_ref[...]
        mx = jnp.max(w3, axis=1, keepdims=True)
        sidx = jnp.min(jnp.where(w3 == mx, siota, c), axis=1, keepdims=True)
        work_ref[...] = jnp.where(siota == sidx, neg, w3)
        cv.append(mx[:, 0, :])
        cg.append(sidx[:, 0, :] * l + liota2)

    def body(i, carry):
        cvs = list(carry[:NLEV])
        cgs = list(carry[NLEV:2 * NLEV])
        colcnt, vacc, iacc = carry[2 * NLEV:]
        cmb = cvs[0]
        for r in range(1, NLEV):
            cmb = jnp.maximum(cmb, cvs[r])
        mx = jnp.max(cmb, axis=1, keepdims=True)
        m8 = jnp.full((b, l), GBIG, jnp.int32)
        for r in range(NLEV):
            m8 = jnp.minimum(m8, jnp.where(cvs[r] == mx, cgs[r], GBIG))
        gmin = jnp.min(m8, axis=1, keepdims=True)
        for r in range(NLEV):
            cvs[r] = jnp.where(cgs[r] == gmin, neg, cvs[r])
        lanecol = gmin - (gmin // l) * l
        colcnt = colcnt + jnp.where(liota2 == lanecol, 1, 0)
        sel = kiota == i
        vacc = jnp.where(sel, mx, vacc)
        iacc = jnp.where(sel, gmin, iacc)
        return tuple(cvs) + tuple(cgs) + (colcnt, vacc, iacc)

    init = tuple(cv) + tuple(cg) + (
        jnp.zeros((b, l), jnp.int32),
        jnp.zeros((b, K), jnp.float32),
        jnp.zeros((b, K), jnp.int32),
    )
    res = lax.fori_loop(0, K, body, init, unroll=8)
    colcnt, vacc, iacc = res[2 * NLEV:]
    gi_ref[...] = iacc + base
    tv_ref[...] = jnp.tanh(vacc)

    overflow = jnp.max(colcnt) >= NLEV

    @pl.when(overflow)
    def _():
        work_ref[...] = s_ref[...] / n_ref[0, 0] + m_ref[...]
        giota = siota * l + lax.broadcasted_iota(jnp.int32, (b, c, l), 2)

        def sbody(i, carry):
            vacc2, iacc2 = carry
            x = work_ref[...]
            mx = jnp.max(jnp.max(x, axis=1, keepdims=True), axis=2,
                         keepdims=True)
            gidx = jnp.where(x == mx, giota, GBIG)
            idxm = jnp.min(jnp.min(gidx, axis=1, keepdims=True), axis=2,
                           keepdims=True)
            work_ref[...] = jnp.where(giota == idxm, neg, x)
            sel = kiota == i
            vacc2 = jnp.where(sel, mx[:, :, 0], vacc2)
            iacc2 = jnp.where(sel, idxm[:, :, 0], iacc2)
            return vacc2, iacc2

        vacc2, iacc2 = lax.fori_loop(
            0, K, sbody,
            (jnp.zeros((b, K), jnp.float32), jnp.zeros((b, K), jnp.int32)),
        )
        gi_ref[...] = iacc2 + base
        tv_ref[...] = jnp.tanh(vacc2)


def _sc_gather(emb_hbm, gi_hbm, out_hbm, idx_v, rows_v, sem):
    num_b = out_hbm.shape[0]
    nc = 2
    wid = lax.axis_index("s") * nc + lax.axis_index("c")

    @pl.when(wid < num_b)
    def _():
        pltpu.sync_copy(gi_hbm.at[wid], idx_v)
        pltpu.async_copy(emb_hbm.at[idx_v], rows_v, sem).wait()
        pltpu.sync_copy(rows_v, out_hbm.at[wid])


def _scale_t_body(x_ref, tv_ref, o_ref):
    xt = jnp.transpose(x_ref[0], (1, 0))
    o_ref[0] = xt * tv_ref[0]


def kernel(node_embs, mask, scorer):
    b, g, f = node_embs.shape
    rows = b * g
    nrm = jnp.linalg.norm(scorer).reshape(1, 1)

    emb_flat = node_embs.reshape(rows, f)
    scores = pl.pallas_call(
        _matvec_body,
        grid=(rows // MV_BLK,),
        in_specs=[
            pl.BlockSpec((MV_BLK, f), lambda i: (i, 0)),
            pl.BlockSpec((f, 1), lambda i: (0, 0)),
        ],
        out_specs=pl.BlockSpec((MV_BLK, 1), lambda i: (i, 0)),
        out_shape=jax.ShapeDtypeStruct((rows, 1), jnp.float32),
    )(emb_flat, scorer)

    nc = g // 128
    gidx, tvals = pl.pallas_call(
        _topk_body,
        in_specs=[
            pl.BlockSpec((b, nc, 128), lambda: (0, 0, 0)),
            pl.BlockSpec((b, nc, 128), lambda: (0, 0, 0)),
            pl.BlockSpec((1, 1), lambda: (0, 0)),
        ],
        out_specs=[
            pl.BlockSpec((b, K), lambda: (0, 0)),
            pl.BlockSpec((b, K), lambda: (0, 0)),
        ],
        out_shape=[
            jax.ShapeDtypeStruct((b, K), jnp.int32),
            jax.ShapeDtypeStruct((b, K), jnp.float32),
        ],
        scratch_shapes=[pltpu.VMEM((b, nc, 128), jnp.float32)],
    )(scores.reshape(b, nc, 128), mask.reshape(b, nc, 128), nrm)

    mesh = plsc.VectorSubcoreMesh(core_axis_name="c", subcore_axis_name="s")
    gathered = pl.kernel(
        _sc_gather,
        out_type=jax.ShapeDtypeStruct((b, K, f), jnp.float32),
        mesh=mesh,
        scratch_types=[
            pltpu.VMEM((K,), jnp.int32),
            pltpu.VMEM((K, f), jnp.float32),
            pltpu.SemaphoreType.DMA,
        ],
    )(emb_flat, gidx)

    out = pl.pallas_call(
        _scale_t_body,
        grid=(b,),
        in_specs=[
            pl.BlockSpec((1, K, f), lambda i: (i, 0, 0)),
            pl.BlockSpec((1, 1, K), lambda i: (i, 0, 0)),
        ],
        out_specs=pl.BlockSpec((1, f, K), lambda i: (i, 0, 0)),
        out_shape=jax.ShapeDtypeStruct((b, f, K), jnp.float32),
    )(gathered, tvals.reshape(b, 1, K))
    return out
